# baseline (speedup 1.0000x reference)
import functools
import math

import jax
import jax.numpy as jnp
from jax.experimental import pallas as pl
from jax.experimental.pallas import tpu as pltpu

KSIZE = 7
DILATIONS = (1, 2, 4, 8, 16)
NUM_SCALES = len(DILATIONS)
MHA_HEADS = 16
GAT_HEADS = 8
LN_EPS = 1e-5
HP = 128
G = 8
GROWS = 256


def _live_taps(T):
    out = []
    for s, dil in enumerate(DILATIONS):
        taps = []
        for j in range(KSIZE):
            shift = (j - 3) * dil
            if abs(shift) < T:
                taps.append((len(taps), j, shift))
        out.append(taps)
    return out


def _softmax_parts(x):
    e = jnp.exp(x)
    return e, 1.0 / jnp.sum(e, axis=-1, keepdims=True)


def _coherence_kernel(T, live, x_ref, wcl_ref, bc_ref, wf_ref,
                      bf_ref, gam_ref, bet_ref, wqkv_ref, bqkv_ref,
                      wo_ref, bo_ref, wg_ref, bg_ref,
                      asrc_ref, adst_ref, h_ref, ssrc_ref, sdst_ref):
    R, D = x_ref.shape[0], x_ref.shape[1]
    X = x_ref[...].astype(jnp.bfloat16)

    rowt = jax.lax.broadcasted_iota(jnp.int32, (R, 1), 0) % T

    z = jnp.zeros((T, D), jnp.float32)
    fused = jnp.zeros((R, D), jnp.float32) + bf_ref[...]
    for s in range(NUM_SCALES):
        acc = jnp.zeros((R, D), jnp.float32) + bc_ref[s]
        for (_pos, j, shift) in live[s]:
            wt = wcl_ref[s * KSIZE + j].astype(jnp.bfloat16)
            Yt = jnp.dot(X, wt, preferred_element_type=jnp.float32)
            Yp = jnp.concatenate([z, Yt, z], axis=0)
            sl = Yp[T + shift:T + shift + R, :]
            t2 = rowt + shift
            valid = (t2 >= 0) & (t2 < T)
            acc = acc + jnp.where(valid, sl, 0.0)
        fused = fused + jnp.dot(acc.astype(jnp.bfloat16),
                                wf_ref[s].astype(jnp.bfloat16),
                                preferred_element_type=jnp.float32)

    mu = jnp.mean(fused, axis=-1, keepdims=True)
    var = jnp.mean(jnp.square(fused - mu), axis=-1, keepdims=True)
    fused = (fused - mu) * jax.lax.rsqrt(var + LN_EPS) * gam_ref[...] + bet_ref[...]
    c = math.sqrt(2.0 / math.pi)
    fused = 0.5 * fused * (1.0 + jnp.tanh(c * (fused + 0.044715 * fused ** 3)))
    fb = fused.astype(jnp.bfloat16)

    hd = D // MHA_HEADS
    scale = 1.0 / math.sqrt(hd)
    DP = MHA_HEADS * HP
    q = (jnp.dot(fb, wqkv_ref[:, 0:DP], preferred_element_type=jnp.float32)
         + bqkv_ref[:, 0:DP])
    k = (jnp.dot(fb, wqkv_ref[:, DP:2 * DP], preferred_element_type=jnp.float32)
         + bqkv_ref[:, DP:2 * DP])
    v = (jnp.dot(fb, wqkv_ref[:, 2 * DP:3 * DP], preferred_element_type=jnp.float32)
         + bqkv_ref[:, 2 * DP:3 * DP])
    qb = (q * scale).astype(jnp.bfloat16)
    kb = k.astype(jnp.bfloat16)
    vb = v.astype(jnp.bfloat16)

    ri = jax.lax.broadcasted_iota(jnp.int32, (R, R), 0) // T
    ci = jax.lax.broadcasted_iota(jnp.int32, (R, R), 1) // T
    maskadd = jnp.where(ri == ci, 0.0, -1e30)

    ohs = []
    for h in range(MHA_HEADS):
        lo = h * HP
        sc = jax.lax.dot_general(qb[:, lo:lo + HP], kb[:, lo:lo + HP],
                                 (((1,), (1,)), ((), ())),
                                 preferred_element_type=jnp.float32)
        e, rz = _softmax_parts(sc + maskadd)
        ohs.append(jnp.dot(e.astype(jnp.bfloat16), vb[:, lo:lo + HP],
                           preferred_element_type=jnp.float32) * rz)
    oh = jnp.concatenate(ohs, axis=-1).astype(jnp.bfloat16)
    att = bo_ref[...] + jnp.dot(oh, wo_ref[...],
                                preferred_element_type=jnp.float32)

    ab = att.astype(jnp.bfloat16)
    hg = jnp.dot(ab, wg_ref[...], preferred_element_type=jnp.float32) + bg_ref[...]
    hgb = hg.astype(jnp.bfloat16)
    h_ref[...] = hgb
    ssrc_ref[...] = jnp.dot(hgb, asrc_ref[...], preferred_element_type=jnp.float32)
    sdst_ref[...] = jnp.dot(hgb, adst_ref[...], preferred_element_type=jnp.float32)


def _gat_kernel(ssrc_ref, sdstT_ref, h_ref, feat_ref, cmat_ref, o_ref):
    Hfull = h_ref[...]
    ss = ssrc_ref[...]
    pieces = []
    for k in range(GAT_HEADS):
        e = ss[:, k:k + 1] + sdstT_ref[k:k + 1, :]
        e = jnp.where(e > 0, e, 0.2 * e)
        ee, rz = _softmax_parts(e)
        pieces.append(jnp.dot(ee.astype(jnp.bfloat16),
                              Hfull[:, k * HP:(k + 1) * HP],
                              preferred_element_type=jnp.float32) * rz)
    hp = jnp.concatenate(pieces, axis=-1)
    elu = jnp.where(hp > 0, hp, jnp.exp(jnp.minimum(hp, 0.0)) - 1.0)
    out = jnp.dot(elu.astype(jnp.bfloat16), cmat_ref[...],
                  preferred_element_type=jnp.float32)
    o_ref[...] = out + feat_ref[...]


def _pad_heads_cols(w, nheads, hd):
    A = w.shape[0]
    return jnp.pad(w.reshape(A, nheads, hd), ((0, 0), (0, 0), (0, HP - hd))
                   ).reshape(A, nheads * HP)


def _pad_heads_rows(w, nheads, hd):
    A = w.shape[1]
    return jnp.pad(w.reshape(nheads, hd, A), ((0, 0), (0, HP - hd), (0, 0))
                   ).reshape(nheads * HP, A)


def kernel(features, wc, bc, wf, bf, gamma, beta, wq, bq, wk, bk, wv, bv,
           wo, bo, wg, bg, a_src_mat, a_dst_mat_t):
    B, T, D = features.shape
    N = B * T
    R = G * T
    live = _live_taps(T)
    hd = D // MHA_HEADS
    hdg = D // GAT_HEADS
    bf16 = jnp.bfloat16

    x2 = features.reshape(N, D)

    ntaps = NUM_SCALES * KSIZE
    wcl = wc
    wfb = wf

    wqkvp = jnp.concatenate(
        [_pad_heads_cols(w_.astype(bf16), MHA_HEADS, hd) for w_ in (wq, wk, wv)],
        axis=1)
    bqkvp = jnp.concatenate(
        [_pad_heads_cols(b_, MHA_HEADS, hd) for b_ in (bq, bk, bv)],
        axis=1)
    wop = _pad_heads_rows(wo.astype(bf16), MHA_HEADS, hd)

    wgp = _pad_heads_cols(wg.astype(bf16), GAT_HEADS, hdg)
    bgp = _pad_heads_cols(bg, GAT_HEADS, hdg)
    asrcp = _pad_heads_rows(a_src_mat, GAT_HEADS, hdg).astype(bf16)
    adstp = _pad_heads_rows(a_dst_mat_t.T, GAT_HEADS, hdg).astype(bf16)

    DP = MHA_HEADS * HP
    DG = GAT_HEADS * HP

    c2 = lambda i: (0, 0)
    c3 = lambda i: (0, 0, 0)
    r2 = lambda i: (i, 0)

    k1 = functools.partial(_coherence_kernel, T, live)
    h_pad, ssrc, sdst = pl.pallas_call(
        k1,
        out_shape=(jax.ShapeDtypeStruct((N, DG), bf16),
                   jax.ShapeDtypeStruct((N, GAT_HEADS), jnp.float32),
                   jax.ShapeDtypeStruct((N, GAT_HEADS), jnp.float32)),
        grid=(B // G,),
        in_specs=[
            pl.BlockSpec((R, D), r2),
            pl.BlockSpec((ntaps, D, D), c3),
            pl.BlockSpec((NUM_SCALES, 1, D), c3),
            pl.BlockSpec((NUM_SCALES, D, D), c3),
            pl.BlockSpec((1, D), c2),
            pl.BlockSpec((1, D), c2),
            pl.BlockSpec((1, D), c2),
            pl.BlockSpec((D, 3 * DP), c2),
            pl.BlockSpec((1, 3 * DP), c2),
            pl.BlockSpec((DP, D), c2),
            pl.BlockSpec((1, D), c2),
            pl.BlockSpec((D, DG), c2),
            pl.BlockSpec((1, DG), c2),
            pl.BlockSpec((DG, GAT_HEADS), c2),
            pl.BlockSpec((DG, GAT_HEADS), c2),
        ],
        out_specs=(pl.BlockSpec((R, DG), r2),
                   pl.BlockSpec((R, GAT_HEADS), r2),
                   pl.BlockSpec((R, GAT_HEADS), r2)),
        compiler_params=pltpu.CompilerParams(
            dimension_semantics=("parallel",),
            vmem_limit_bytes=60000 * 1024),
    )(x2, wcl, bc, wfb, bf, gamma, beta, wqkvp, bqkvp,
      wop, bo, wgp, bgp, asrcp, adstp)

    sdstT = sdst.T
    eyeh = jnp.eye(hdg, dtype=bf16)
    cmat = jnp.pad(eyeh[None], ((0, 0), (0, HP - hdg), (0, 0)))
    cmat = jax.scipy.linalg.block_diag(
        *[cmat[0]] * GAT_HEADS).astype(bf16)

    out_flat = pl.pallas_call(
        _gat_kernel,
        out_shape=jax.ShapeDtypeStruct((N, D), jnp.float32),
        grid=(N // GROWS,),
        in_specs=[
            pl.BlockSpec((GROWS, GAT_HEADS), r2),
            pl.BlockSpec((GAT_HEADS, N), c2),
            pl.BlockSpec((N, DG), c2),
            pl.BlockSpec((GROWS, D), r2),
            pl.BlockSpec((DG, D), c2),
        ],
        out_specs=pl.BlockSpec((GROWS, D), r2),
        compiler_params=pltpu.CompilerParams(
            dimension_semantics=("parallel",),
            vmem_limit_bytes=60000 * 1024),
    )(ssrc, sdstT, h_pad, x2, cmat)

    return out_flat.reshape(B, T, D)


# speedup vs baseline: 1.0262x; 1.0262x over previous
import functools
import math

import jax
import jax.numpy as jnp
from jax.experimental import pallas as pl
from jax.experimental.pallas import tpu as pltpu

KSIZE = 7
DILATIONS = (1, 2, 4, 8, 16)
NUM_SCALES = len(DILATIONS)
MHA_HEADS = 16
GAT_HEADS = 8
LN_EPS = 1e-5
HP = 128
G = 16
GROWS = 512


def _live_taps(T):
    out = []
    for s, dil in enumerate(DILATIONS):
        taps = []
        for j in range(KSIZE):
            shift = (j - 3) * dil
            if abs(shift) < T:
                taps.append((len(taps), j, shift))
        out.append(taps)
    return out


def _softmax_parts(x):
    e = jnp.exp(x)
    return e, 1.0 / jnp.sum(e, axis=-1, keepdims=True)


def _coherence_kernel(T, live, x_ref, wcl_ref, bc_ref, wf_ref,
                      bf_ref, gam_ref, bet_ref, wqkv_ref, bqkv_ref,
                      wo_ref, bo_ref, wg_ref, bg_ref,
                      asrc_ref, adst_ref, h_ref, ssrc_ref, sdst_ref):
    R, D = x_ref.shape[0], x_ref.shape[1]
    X = x_ref[...].astype(jnp.bfloat16)

    rowt = jax.lax.broadcasted_iota(jnp.int32, (R, 1), 0) % T

    z = jnp.zeros((T, D), jnp.float32)
    fused = jnp.zeros((R, D), jnp.float32) + bf_ref[...]
    for s in range(NUM_SCALES):
        acc = jnp.zeros((R, D), jnp.float32) + bc_ref[s]
        for (_pos, j, shift) in live[s]:
            wt = wcl_ref[s * KSIZE + j].astype(jnp.bfloat16)
            Yt = jnp.dot(X, wt, preferred_element_type=jnp.float32)
            Yp = jnp.concatenate([z, Yt, z], axis=0)
            sl = Yp[T + shift:T + shift + R, :]
            t2 = rowt + shift
            valid = (t2 >= 0) & (t2 < T)
            acc = acc + jnp.where(valid, sl, 0.0)
        fused = fused + jnp.dot(acc.astype(jnp.bfloat16),
                                wf_ref[s].astype(jnp.bfloat16),
                                preferred_element_type=jnp.float32)

    mu = jnp.mean(fused, axis=-1, keepdims=True)
    var = jnp.mean(jnp.square(fused - mu), axis=-1, keepdims=True)
    fused = (fused - mu) * jax.lax.rsqrt(var + LN_EPS) * gam_ref[...] + bet_ref[...]
    c = math.sqrt(2.0 / math.pi)
    fused = 0.5 * fused * (1.0 + jnp.tanh(c * (fused + 0.044715 * fused ** 3)))
    fb = fused.astype(jnp.bfloat16)

    hd = D // MHA_HEADS
    scale = 1.0 / math.sqrt(hd)
    DP = MHA_HEADS * HP
    q = (jnp.dot(fb, wqkv_ref[:, 0:DP], preferred_element_type=jnp.float32)
         + bqkv_ref[:, 0:DP])
    k = (jnp.dot(fb, wqkv_ref[:, DP:2 * DP], preferred_element_type=jnp.float32)
         + bqkv_ref[:, DP:2 * DP])
    v = (jnp.dot(fb, wqkv_ref[:, 2 * DP:3 * DP], preferred_element_type=jnp.float32)
         + bqkv_ref[:, 2 * DP:3 * DP])
    qb = (q * scale).astype(jnp.bfloat16)
    kb = k.astype(jnp.bfloat16)
    vb = v.astype(jnp.bfloat16)

    ri = jax.lax.broadcasted_iota(jnp.int32, (R, R), 0) // T
    ci = jax.lax.broadcasted_iota(jnp.int32, (R, R), 1) // T
    maskadd = jnp.where(ri == ci, 0.0, -1e30)

    ohs = []
    for h in range(MHA_HEADS):
        lo = h * HP
        sc = jax.lax.dot_general(qb[:, lo:lo + HP], kb[:, lo:lo + HP],
                                 (((1,), (1,)), ((), ())),
                                 preferred_element_type=jnp.float32)
        e, rz = _softmax_parts(sc + maskadd)
        ohs.append(jnp.dot(e.astype(jnp.bfloat16), vb[:, lo:lo + HP],
                           preferred_element_type=jnp.float32) * rz)
    oh = jnp.concatenate(ohs, axis=-1).astype(jnp.bfloat16)
    att = bo_ref[...] + jnp.dot(oh, wo_ref[...],
                                preferred_element_type=jnp.float32)

    ab = att.astype(jnp.bfloat16)
    hg = jnp.dot(ab, wg_ref[...], preferred_element_type=jnp.float32) + bg_ref[...]
    hgb = hg.astype(jnp.bfloat16)
    h_ref[...] = hgb
    ssrc_ref[...] = jnp.dot(hgb, asrc_ref[...], preferred_element_type=jnp.float32)
    sdst_ref[...] = jnp.dot(hgb, adst_ref[...], preferred_element_type=jnp.float32)


def _gat_kernel(ssrc_ref, sdstT_ref, h_ref, feat_ref, cmat_ref, o_ref):
    Hfull = h_ref[...]
    ss = ssrc_ref[...]
    pieces = []
    for k in range(GAT_HEADS):
        e = ss[:, k:k + 1] + sdstT_ref[k:k + 1, :]
        e = jnp.where(e > 0, e, 0.2 * e)
        ee, rz = _softmax_parts(e)
        pieces.append(jnp.dot(ee.astype(jnp.bfloat16),
                              Hfull[:, k * HP:(k + 1) * HP],
                              preferred_element_type=jnp.float32) * rz)
    hp = jnp.concatenate(pieces, axis=-1)
    elu = jnp.where(hp > 0, hp, jnp.exp(jnp.minimum(hp, 0.0)) - 1.0)
    out = jnp.dot(elu.astype(jnp.bfloat16), cmat_ref[...],
                  preferred_element_type=jnp.float32)
    o_ref[...] = out + feat_ref[...]


def _pad_heads_cols(w, nheads, hd):
    A = w.shape[0]
    return jnp.pad(w.reshape(A, nheads, hd), ((0, 0), (0, 0), (0, HP - hd))
                   ).reshape(A, nheads * HP)


def _pad_heads_rows(w, nheads, hd):
    A = w.shape[1]
    return jnp.pad(w.reshape(nheads, hd, A), ((0, 0), (0, HP - hd), (0, 0))
                   ).reshape(nheads * HP, A)


def kernel(features, wc, bc, wf, bf, gamma, beta, wq, bq, wk, bk, wv, bv,
           wo, bo, wg, bg, a_src_mat, a_dst_mat_t):
    B, T, D = features.shape
    N = B * T
    R = G * T
    live = _live_taps(T)
    hd = D // MHA_HEADS
    hdg = D // GAT_HEADS
    bf16 = jnp.bfloat16

    x2 = features.reshape(N, D)

    ntaps = NUM_SCALES * KSIZE
    wcl = wc
    wfb = wf

    wqkvp = jnp.concatenate(
        [_pad_heads_cols(w_.astype(bf16), MHA_HEADS, hd) for w_ in (wq, wk, wv)],
        axis=1)
    bqkvp = jnp.concatenate(
        [_pad_heads_cols(b_, MHA_HEADS, hd) for b_ in (bq, bk, bv)],
        axis=1)
    wop = _pad_heads_rows(wo.astype(bf16), MHA_HEADS, hd)

    wgp = _pad_heads_cols(wg.astype(bf16), GAT_HEADS, hdg)
    bgp = _pad_heads_cols(bg, GAT_HEADS, hdg)
    asrcp = _pad_heads_rows(a_src_mat, GAT_HEADS, hdg).astype(bf16)
    adstp = _pad_heads_rows(a_dst_mat_t.T, GAT_HEADS, hdg).astype(bf16)

    DP = MHA_HEADS * HP
    DG = GAT_HEADS * HP

    c2 = lambda i: (0, 0)
    c3 = lambda i: (0, 0, 0)
    r2 = lambda i: (i, 0)

    k1 = functools.partial(_coherence_kernel, T, live)
    h_pad, ssrc, sdst = pl.pallas_call(
        k1,
        out_shape=(jax.ShapeDtypeStruct((N, DG), bf16),
                   jax.ShapeDtypeStruct((N, GAT_HEADS), jnp.float32),
                   jax.ShapeDtypeStruct((N, GAT_HEADS), jnp.float32)),
        grid=(B // G,),
        in_specs=[
            pl.BlockSpec((R, D), r2),
            pl.BlockSpec((ntaps, D, D), c3),
            pl.BlockSpec((NUM_SCALES, 1, D), c3),
            pl.BlockSpec((NUM_SCALES, D, D), c3),
            pl.BlockSpec((1, D), c2),
            pl.BlockSpec((1, D), c2),
            pl.BlockSpec((1, D), c2),
            pl.BlockSpec((D, 3 * DP), c2),
            pl.BlockSpec((1, 3 * DP), c2),
            pl.BlockSpec((DP, D), c2),
            pl.BlockSpec((1, D), c2),
            pl.BlockSpec((D, DG), c2),
            pl.BlockSpec((1, DG), c2),
            pl.BlockSpec((DG, GAT_HEADS), c2),
            pl.BlockSpec((DG, GAT_HEADS), c2),
        ],
        out_specs=(pl.BlockSpec((R, DG), r2),
                   pl.BlockSpec((R, GAT_HEADS), r2),
                   pl.BlockSpec((R, GAT_HEADS), r2)),
        compiler_params=pltpu.CompilerParams(
            dimension_semantics=("parallel",),
            vmem_limit_bytes=60000 * 1024),
    )(x2, wcl, bc, wfb, bf, gamma, beta, wqkvp, bqkvp,
      wop, bo, wgp, bgp, asrcp, adstp)

    sdstT = sdst.T
    eyeh = jnp.eye(hdg, dtype=bf16)
    cmat = jnp.pad(eyeh[None], ((0, 0), (0, HP - hdg), (0, 0)))
    cmat = jax.scipy.linalg.block_diag(
        *[cmat[0]] * GAT_HEADS).astype(bf16)

    out_flat = pl.pallas_call(
        _gat_kernel,
        out_shape=jax.ShapeDtypeStruct((N, D), jnp.float32),
        grid=(N // GROWS,),
        in_specs=[
            pl.BlockSpec((GROWS, GAT_HEADS), r2),
            pl.BlockSpec((GAT_HEADS, N), c2),
            pl.BlockSpec((N, DG), c2),
            pl.BlockSpec((GROWS, D), r2),
            pl.BlockSpec((DG, D), c2),
        ],
        out_specs=pl.BlockSpec((GROWS, D), r2),
        compiler_params=pltpu.CompilerParams(
            dimension_semantics=("parallel",),
            vmem_limit_bytes=60000 * 1024),
    )(ssrc, sdstT, h_pad, x2, cmat)

    return out_flat.reshape(B, T, D)
